# Initial kernel scaffold; baseline (speedup 1.0000x reference)
import jax
import jax.numpy as jnp
from jax.experimental import pallas as pl


def kernel(states, action, viz, eW1, eb1, eW2, eb2, eg, ebeta, eW3, eb3, nW1, nb1, nW2, nb2, ng, nbeta, nW3, nb3):
    raise NotImplementedError("write your pallas kernel here")


# speedup vs baseline: 24.3076x; 24.3076x over previous
import functools

import jax
import jax.numpy as jnp
from jax.experimental import pallas as pl

_B, _N, _D_IN, _D_H, _D_OUT = 256, 64, 32, 64, 32
_BT = 8


def _ln(x, g, b, eps=1e-5):
    mu = jnp.mean(x, axis=-1, keepdims=True)
    var = jnp.mean((x - mu) ** 2, axis=-1, keepdims=True)
    return (x - mu) * jax.lax.rsqrt(var + eps) * g + b


def _fused_kernel(x_ref, eW1a_ref, eW1b_ref, eb1_ref, eW2_ref, eb2_ref,
                  eg_ref, ebeta_ref, eW3_ref, eb3_ref,
                  nW1_ref, nb1_ref, nW2_ref, nb2_ref, ng_ref, nbeta_ref,
                  nW3_ref, nb3_ref, out_ref):
    x = x_ref[...]
    xf = x.reshape(_BT * _N, _D_IN)
    p = xf @ eW1a_ref[...]
    q = xf @ eW1b_ref[...] + eb1_ref[...]
    p = p.reshape(_BT, _N, 1, _D_H)
    q = q.reshape(_BT, 1, _N, _D_H)
    h1 = jnp.maximum(p + q, 0.0).reshape(_BT * _N * _N, _D_H)
    h2 = h1 @ eW2_ref[...] + eb2_ref[...]
    h2 = jnp.maximum(_ln(h2, eg_ref[...], ebeta_ref[...]), 0.0)
    ea = (h2 @ eW3_ref[...] + eb3_ref[...]).reshape(_BT, _N, _N, _D_H)
    ii = jax.lax.broadcasted_iota(jnp.int32, (_N, _N), 0)
    jj = jax.lax.broadcasted_iota(jnp.int32, (_N, _N), 1)
    mask = (ii != jj).astype(jnp.float32)[None, :, :, None]
    agg = jnp.sum(ea * mask, axis=2)
    nin = jnp.concatenate([x, agg], axis=-1).reshape(_BT * _N, _D_IN + _D_H)
    g1 = jnp.maximum(nin @ nW1_ref[...] + nb1_ref[...], 0.0)
    g2 = g1 @ nW2_ref[...] + nb2_ref[...]
    g2 = jnp.maximum(_ln(g2, ng_ref[...], nbeta_ref[...]), 0.0)
    out = g2 @ nW3_ref[...] + nb3_ref[...]
    out_ref[...] = out.reshape(_BT, _N, _D_OUT)


@functools.partial(jax.jit, static_argnames=("interpret",))
def _run(states, eW1, eb1, eW2, eb2, eg, ebeta, eW3, eb3,
         nW1, nb1, nW2, nb2, ng, nbeta, nW3, nb3, interpret=False):
    eW1a, eW1b = eW1[:_D_IN], eW1[_D_IN:]
    row = lambda v: v.reshape(1, -1)
    weights = (eW1a, eW1b, row(eb1), eW2, row(eb2), row(eg), row(ebeta),
               eW3, row(eb3), nW1, row(nb1), nW2, row(nb2), row(ng),
               row(nbeta), nW3, row(nb3))
    full = lambda w: pl.BlockSpec(w.shape, lambda b: (0,) * w.ndim)
    grid = _B // _BT
    out = pl.pallas_call(
        _fused_kernel,
        grid=(grid,),
        in_specs=[pl.BlockSpec((_BT, _N, _D_IN), lambda b: (b, 0, 0))]
                 + [full(w) for w in weights],
        out_specs=pl.BlockSpec((_BT, _N, _D_OUT), lambda b: (b, 0, 0)),
        out_shape=jax.ShapeDtypeStruct((_B, _N, _D_OUT), jnp.float32),
        interpret=interpret,
    )(states, *weights)
    return out


def kernel(states, action, viz, eW1, eb1, eW2, eb2, eg, ebeta, eW3, eb3,
           nW1, nb1, nW2, nb2, ng, nbeta, nW3, nb3):
    out = _run(states, eW1, eb1, eW2, eb2, eg, ebeta, eW3, eb3,
               nW1, nb1, nW2, nb2, ng, nbeta, nW3, nb3)
    return (out, action, viz)


# speedup vs baseline: 44.8634x; 1.8457x over previous
import functools

import jax
import jax.numpy as jnp
from jax.experimental import pallas as pl

_B, _N, _D_IN, _D_H, _D_OUT = 256, 64, 32, 64, 32
_BT = 8
_EPS = 1e-5


def _ln64(x, g, b):
    mu = jnp.mean(x, axis=-1, keepdims=True)
    var = jnp.mean((x - mu) ** 2, axis=-1, keepdims=True)
    return (x - mu) * jax.lax.rsqrt(var + _EPS) * g + b


def _fused_kernel(x_ref, eW1a2_ref, eW1a_ref, eW1b_ref, eb1_ref,
                  W2d_ref, b2d_ref, W2A_ref, b2A_ref, A_ref, gd_ref,
                  betad_ref, W3v_ref, eb3row_ref,
                  eW2_ref, eb2_ref, eg_ref, ebeta_ref, eW3_ref,
                  nW1_ref, nb1_ref, nW2_ref, nb2_ref, ng_ref, nbeta_ref,
                  nW3_ref, nb3_ref, out_ref):
    x = x_ref[...]
    xf = x.reshape(_BT * _N, _D_IN)
    p2 = xf @ eW1a2_ref[...]
    qs = xf @ eW1b_ref[...] + eb1_ref[...]
    qlo = x[:, :_N // 2, :].reshape(_BT * (_N // 2), _D_IN)
    qhi = x[:, _N // 2:, :].reshape(_BT * (_N // 2), _D_IN)
    q2 = jnp.concatenate([qlo @ eW1b_ref[...] + eb1_ref[...],
                          qhi @ eW1b_ref[...] + eb1_ref[...]], axis=-1)
    q2 = q2.reshape(_BT, _N // 2, 2 * _D_H)
    h1 = jnp.maximum(p2.reshape(_BT, _N, 1, 2 * _D_H) + q2[:, None], 0.0)
    h1 = h1.reshape(_BT * _N * (_N // 2), 2 * _D_H)
    h2 = h1 @ W2d_ref[...] + b2d_ref[...]
    mu = h1 @ W2A_ref[...] + b2A_ref[...]
    d = h2 - mu
    var = (d * d) @ A_ref[...]
    h3 = jnp.maximum(d * jax.lax.rsqrt(var + _EPS) * gd_ref[...]
                     + betad_ref[...], 0.0)
    s = jnp.sum(h3.reshape(_BT, _N, _N // 2, 2 * _D_H), axis=2)
    agg_full = s.reshape(_BT * _N, 2 * _D_H) @ W3v_ref[...] \
        + _N * eb3row_ref[...]
    pd = xf @ eW1a_ref[...]
    d1 = jnp.maximum(pd + qs, 0.0)
    d2 = _ln64(d1 @ eW2_ref[...] + eb2_ref[...], eg_ref[...], ebeta_ref[...])
    d3 = jnp.maximum(d2, 0.0) @ eW3_ref[...] + eb3row_ref[...]
    agg = agg_full - d3
    nin = jnp.concatenate([xf, agg], axis=-1)
    g1 = jnp.maximum(nin @ nW1_ref[...] + nb1_ref[...], 0.0)
    g2 = _ln64(g1 @ nW2_ref[...] + nb2_ref[...], ng_ref[...], nbeta_ref[...])
    out = jnp.maximum(g2, 0.0) @ nW3_ref[...] + nb3_ref[...]
    out_ref[...] = out.reshape(_BT, _N, _D_OUT)


@functools.partial(jax.jit, static_argnames=("interpret",))
def _run(states, eW1, eb1, eW2, eb2, eg, ebeta, eW3, eb3,
         nW1, nb1, nW2, nb2, ng, nbeta, nW3, nb3, interpret=False):
    f32 = jnp.float32
    row = lambda v: v.reshape(1, -1)
    eW1a, eW1b = eW1[:_D_IN], eW1[_D_IN:]
    eW1a2 = jnp.concatenate([eW1a, eW1a], axis=1)
    z = jnp.zeros((_D_H, _D_H), f32)
    W2d = jnp.block([[eW2, z], [z, eW2]])
    b2d = row(jnp.concatenate([eb2, eb2]))
    ones = jnp.ones((_D_H, _D_H), f32) / _D_H
    A = jnp.block([[ones, z], [z, ones]])
    W2A = W2d @ A
    b2A = b2d @ A
    gd = row(jnp.concatenate([eg, eg]))
    betad = row(jnp.concatenate([ebeta, ebeta]))
    W3v = jnp.concatenate([eW3, eW3], axis=0)
    weights = (eW1a2, eW1a, eW1b, row(eb1), W2d, b2d, W2A, b2A, A, gd,
               betad, W3v, row(eb3), eW2, row(eb2), row(eg), row(ebeta),
               eW3, nW1, row(nb1), nW2, row(nb2), row(ng), row(nbeta),
               nW3, row(nb3))
    full = lambda w: pl.BlockSpec(w.shape, lambda b: (0,) * w.ndim)
    out = pl.pallas_call(
        _fused_kernel,
        grid=(_B // _BT,),
        in_specs=[pl.BlockSpec((_BT, _N, _D_IN), lambda b: (b, 0, 0))]
                 + [full(w) for w in weights],
        out_specs=pl.BlockSpec((_BT, _N, _D_OUT), lambda b: (b, 0, 0)),
        out_shape=jax.ShapeDtypeStruct((_B, _N, _D_OUT), f32),
        interpret=interpret,
    )(states, *weights)
    return out


def kernel(states, action, viz, eW1, eb1, eW2, eb2, eg, ebeta, eW3, eb3,
           nW1, nb1, nW2, nb2, ng, nbeta, nW3, nb3):
    out = _run(states, eW1, eb1, eW2, eb2, eg, ebeta, eW3, eb3,
               nW1, nb1, nW2, nb2, ng, nbeta, nW3, nb3)
    return (out, action, viz)


# speedup vs baseline: 61.8732x; 1.3791x over previous
import functools

import jax
import jax.numpy as jnp
from jax.experimental import pallas as pl
from jax.experimental.pallas import tpu as pltpu

_B, _N, _D_IN, _D_H, _D_OUT = 256, 64, 32, 64, 32
_BT = 8
_EPS = 1e-5


def _ln64(x, g, b):
    mu = jnp.mean(x, axis=-1, keepdims=True)
    var = jnp.mean((x - mu) ** 2, axis=-1, keepdims=True)
    return (x - mu) * jax.lax.rsqrt(var + _EPS) * g + b


def _fused_kernel(x_ref, eW1a2_ref, eW1a_ref, eW1b_ref, eb1_ref,
                  Wc_ref, bc_ref, Ag_ref,
                  betad_ref, W3v_ref, eb3row_ref,
                  eW2_ref, eb2_ref, eg_ref, ebeta_ref, eW3_ref,
                  nW1_ref, nb1_ref, nW2_ref, nb2_ref, ng_ref, nbeta_ref,
                  nW3_ref, nb3_ref, out_ref):
    x = x_ref[...]
    xf = x.reshape(_BT * _N, _D_IN)
    p2 = xf @ eW1a2_ref[...]
    qs = xf @ eW1b_ref[...] + eb1_ref[...]
    qlo = x[:, :_N // 2, :].reshape(_BT * (_N // 2), _D_IN)
    qhi = x[:, _N // 2:, :].reshape(_BT * (_N // 2), _D_IN)
    q2 = jnp.concatenate([qlo @ eW1b_ref[...] + eb1_ref[...],
                          qhi @ eW1b_ref[...] + eb1_ref[...]], axis=-1)
    q2 = q2.reshape(_BT, _N // 2, 2 * _D_H)
    h1 = jnp.maximum(p2.reshape(_BT, _N, 1, 2 * _D_H) + q2[:, None], 0.0)
    h1 = h1.reshape(_BT * _N * (_N // 2), 2 * _D_H)
    d = h1 @ Wc_ref[...] + bc_ref[...]
    var = (d * d) @ Ag_ref[...]
    h3 = jnp.maximum(d * jax.lax.rsqrt(var + _EPS) + betad_ref[...], 0.0)
    s = jnp.sum(h3.reshape(_BT, _N, _N // 2, 2 * _D_H), axis=2)
    agg_full = s.reshape(_BT * _N, 2 * _D_H) @ W3v_ref[...] \
        + _N * eb3row_ref[...]
    pd = xf @ eW1a_ref[...]
    d1 = jnp.maximum(pd + qs, 0.0)
    d2 = _ln64(d1 @ eW2_ref[...] + eb2_ref[...], eg_ref[...], ebeta_ref[...])
    d3 = jnp.maximum(d2, 0.0) @ eW3_ref[...] + eb3row_ref[...]
    agg = agg_full - d3
    nin = jnp.concatenate([xf, agg], axis=-1)
    g1 = jnp.maximum(nin @ nW1_ref[...] + nb1_ref[...], 0.0)
    g2 = _ln64(g1 @ nW2_ref[...] + nb2_ref[...], ng_ref[...], nbeta_ref[...])
    out = jnp.maximum(g2, 0.0) @ nW3_ref[...] + nb3_ref[...]
    out_ref[...] = out.reshape(_BT, _N, _D_OUT)


@functools.partial(jax.jit, static_argnames=("interpret",))
def _run(states, eW1, eb1, eW2, eb2, eg, ebeta, eW3, eb3,
         nW1, nb1, nW2, nb2, ng, nbeta, nW3, nb3, interpret=False):
    f32 = jnp.float32
    row = lambda v: v.reshape(1, -1)
    eW1a, eW1b = eW1[:_D_IN], eW1[_D_IN:]
    eW1a2 = jnp.concatenate([eW1a, eW1a], axis=1)
    z = jnp.zeros((_D_H, _D_H), f32)
    W2d = jnp.block([[eW2, z], [z, eW2]])
    b2d = row(jnp.concatenate([eb2, eb2]))
    ones = jnp.ones((_D_H, _D_H), f32) / _D_H
    A = jnp.block([[ones, z], [z, ones]])
    gd = jnp.concatenate([eg, eg])
    Wc = (W2d - W2d @ A) * gd[None, :]
    bc = (b2d - b2d @ A) * gd[None, :]
    Ag = A / (gd * gd)[:, None]
    betad = row(jnp.concatenate([ebeta, ebeta]))
    W3v = jnp.concatenate([eW3, eW3], axis=0)
    weights = (eW1a2, eW1a, eW1b, row(eb1), Wc, bc, Ag,
               betad, W3v, row(eb3), eW2, row(eb2), row(eg), row(ebeta),
               eW3, nW1, row(nb1), nW2, row(nb2), row(ng), row(nbeta),
               nW3, row(nb3))
    full = lambda w: pl.BlockSpec(w.shape, lambda b: (0,) * w.ndim)
    out = pl.pallas_call(
        _fused_kernel,
        grid=(_B // _BT,),
        in_specs=[pl.BlockSpec((_BT, _N, _D_IN), lambda b: (b, 0, 0))]
                 + [full(w) for w in weights],
        out_specs=pl.BlockSpec((_BT, _N, _D_OUT), lambda b: (b, 0, 0)),
        out_shape=jax.ShapeDtypeStruct((_B, _N, _D_OUT), f32),
        compiler_params=pltpu.CompilerParams(
            dimension_semantics=("parallel",)),
        interpret=interpret,
    )(states, *weights)
    return out


def kernel(states, action, viz, eW1, eb1, eW2, eb2, eg, ebeta, eW3, eb3,
           nW1, nb1, nW2, nb2, ng, nbeta, nW3, nb3):
    out = _run(states, eW1, eb1, eW2, eb2, eg, ebeta, eW3, eb3,
               nW1, nb1, nW2, nb2, ng, nbeta, nW3, nb3)
    return (out, action, viz)


# speedup vs baseline: 69.5097x; 1.1234x over previous
import functools

import jax
import jax.numpy as jnp
from jax.experimental import pallas as pl
from jax.experimental.pallas import tpu as pltpu

_B, _N, _D_IN, _D_H, _D_OUT = 256, 64, 32, 64, 32
_BT = 16
_EPS = 1e-5


def _ln64(x, g, b):
    mu = jnp.mean(x, axis=-1, keepdims=True)
    var = jnp.mean((x - mu) ** 2, axis=-1, keepdims=True)
    return (x - mu) * jax.lax.rsqrt(var + _EPS) * g + b


def _fused_kernel(x_ref, eW1a2_ref, eW1a_ref, eW1b_ref, eb1_ref,
                  Wc_ref, bc_ref, Ag_ref,
                  betad_ref, W3v_ref, eb3row_ref,
                  eW2_ref, eb2_ref, eg_ref, ebeta_ref, eW3_ref,
                  nW1_ref, nb1_ref, nW2_ref, nb2_ref, ng_ref, nbeta_ref,
                  nW3_ref, nb3_ref, out_ref):
    x = x_ref[...]
    xf = x.reshape(_BT * _N, _D_IN)
    p2 = xf @ eW1a2_ref[...]
    qs = xf @ eW1b_ref[...] + eb1_ref[...]
    qlo = x[:, :_N // 2, :].reshape(_BT * (_N // 2), _D_IN)
    qhi = x[:, _N // 2:, :].reshape(_BT * (_N // 2), _D_IN)
    q2 = jnp.concatenate([qlo @ eW1b_ref[...] + eb1_ref[...],
                          qhi @ eW1b_ref[...] + eb1_ref[...]], axis=-1)
    q2 = q2.reshape(_BT, _N // 2, 2 * _D_H)
    h1 = jnp.maximum(p2.reshape(_BT, _N, 1, 2 * _D_H) + q2[:, None], 0.0)
    h1 = h1.reshape(_BT * _N * (_N // 2), 2 * _D_H)
    d = h1 @ Wc_ref[...] + bc_ref[...]
    var = (d * d) @ Ag_ref[...]
    h3 = jnp.maximum(d * jax.lax.rsqrt(var + _EPS) + betad_ref[...], 0.0)
    s = jnp.sum(h3.reshape(_BT, _N, _N // 2, 2 * _D_H), axis=2)
    agg_full = s.reshape(_BT * _N, 2 * _D_H) @ W3v_ref[...] \
        + _N * eb3row_ref[...]
    pd = xf @ eW1a_ref[...]
    d1 = jnp.maximum(pd + qs, 0.0)
    d2 = _ln64(d1 @ eW2_ref[...] + eb2_ref[...], eg_ref[...], ebeta_ref[...])
    d3 = jnp.maximum(d2, 0.0) @ eW3_ref[...] + eb3row_ref[...]
    agg = agg_full - d3
    nin = jnp.concatenate([xf, agg], axis=-1)
    g1 = jnp.maximum(nin @ nW1_ref[...] + nb1_ref[...], 0.0)
    g2 = _ln64(g1 @ nW2_ref[...] + nb2_ref[...], ng_ref[...], nbeta_ref[...])
    out = jnp.maximum(g2, 0.0) @ nW3_ref[...] + nb3_ref[...]
    out_ref[...] = out.reshape(_BT, _N, _D_OUT)


@functools.partial(jax.jit, static_argnames=("interpret",))
def _run(states, eW1, eb1, eW2, eb2, eg, ebeta, eW3, eb3,
         nW1, nb1, nW2, nb2, ng, nbeta, nW3, nb3, interpret=False):
    f32 = jnp.float32
    row = lambda v: v.reshape(1, -1)
    eW1a, eW1b = eW1[:_D_IN], eW1[_D_IN:]
    eW1a2 = jnp.concatenate([eW1a, eW1a], axis=1)
    z = jnp.zeros((_D_H, _D_H), f32)
    W2d = jnp.block([[eW2, z], [z, eW2]])
    b2d = row(jnp.concatenate([eb2, eb2]))
    ones = jnp.ones((_D_H, _D_H), f32) / _D_H
    A = jnp.block([[ones, z], [z, ones]])
    gd = jnp.concatenate([eg, eg])
    Wc = (W2d - W2d @ A) * gd[None, :]
    bc = (b2d - b2d @ A) * gd[None, :]
    Ag = A / (gd * gd)[:, None]
    betad = row(jnp.concatenate([ebeta, ebeta]))
    W3v = jnp.concatenate([eW3, eW3], axis=0)
    weights = (eW1a2, eW1a, eW1b, row(eb1), Wc, bc, Ag,
               betad, W3v, row(eb3), eW2, row(eb2), row(eg), row(ebeta),
               eW3, nW1, row(nb1), nW2, row(nb2), row(ng), row(nbeta),
               nW3, row(nb3))
    full = lambda w: pl.BlockSpec(w.shape, lambda b: (0,) * w.ndim)
    out = pl.pallas_call(
        _fused_kernel,
        grid=(_B // _BT,),
        in_specs=[pl.BlockSpec((_BT, _N, _D_IN), lambda b: (b, 0, 0))]
                 + [full(w) for w in weights],
        out_specs=pl.BlockSpec((_BT, _N, _D_OUT), lambda b: (b, 0, 0)),
        out_shape=jax.ShapeDtypeStruct((_B, _N, _D_OUT), f32),
        compiler_params=pltpu.CompilerParams(
            dimension_semantics=("parallel",)),
        interpret=interpret,
    )(states, *weights)
    return out


def kernel(states, action, viz, eW1, eb1, eW2, eb2, eg, ebeta, eW3, eb3,
           nW1, nb1, nW2, nb2, ng, nbeta, nW3, nb3):
    out = _run(states, eW1, eb1, eW2, eb2, eg, ebeta, eW3, eb3,
               nW1, nb1, nW2, nb2, ng, nbeta, nW3, nb3)
    return (out, action, viz)


# speedup vs baseline: 70.3397x; 1.0119x over previous
import functools

import jax
import jax.numpy as jnp
from jax.experimental import pallas as pl
from jax.experimental.pallas import tpu as pltpu

_B, _N, _D_IN, _D_H, _D_OUT = 256, 64, 32, 64, 32
_BT = 32
_EPS = 1e-5


def _ln64(x, g, b):
    mu = jnp.mean(x, axis=-1, keepdims=True)
    var = jnp.mean((x - mu) ** 2, axis=-1, keepdims=True)
    return (x - mu) * jax.lax.rsqrt(var + _EPS) * g + b


def _fused_kernel(x_ref, eW1a2_ref, eW1a_ref, eW1b_ref, eb1_ref,
                  Wc_ref, bc_ref, Ag_ref,
                  betad_ref, W3v_ref, eb3row_ref,
                  eW2_ref, eb2_ref, eg_ref, ebeta_ref, eW3_ref,
                  nW1_ref, nb1_ref, nW2_ref, nb2_ref, ng_ref, nbeta_ref,
                  nW3_ref, nb3_ref, out_ref):
    x = x_ref[...]
    xf = x.reshape(_BT * _N, _D_IN)
    p2 = xf @ eW1a2_ref[...]
    qs = xf @ eW1b_ref[...] + eb1_ref[...]
    qlo = x[:, :_N // 2, :].reshape(_BT * (_N // 2), _D_IN)
    qhi = x[:, _N // 2:, :].reshape(_BT * (_N // 2), _D_IN)
    q2 = jnp.concatenate([qlo @ eW1b_ref[...] + eb1_ref[...],
                          qhi @ eW1b_ref[...] + eb1_ref[...]], axis=-1)
    q2 = q2.reshape(_BT, _N // 2, 2 * _D_H)
    h1 = jnp.maximum(p2.reshape(_BT, _N, 1, 2 * _D_H) + q2[:, None], 0.0)
    h1 = h1.reshape(_BT * _N * (_N // 2), 2 * _D_H)
    d = h1 @ Wc_ref[...] + bc_ref[...]
    var = (d * d) @ Ag_ref[...]
    h3 = jnp.maximum(d * jax.lax.rsqrt(var + _EPS) + betad_ref[...], 0.0)
    s = jnp.sum(h3.reshape(_BT, _N, _N // 2, 2 * _D_H), axis=2)
    agg_full = s.reshape(_BT * _N, 2 * _D_H) @ W3v_ref[...] \
        + _N * eb3row_ref[...]
    pd = xf @ eW1a_ref[...]
    d1 = jnp.maximum(pd + qs, 0.0)
    d2 = _ln64(d1 @ eW2_ref[...] + eb2_ref[...], eg_ref[...], ebeta_ref[...])
    d3 = jnp.maximum(d2, 0.0) @ eW3_ref[...] + eb3row_ref[...]
    agg = agg_full - d3
    nin = jnp.concatenate([xf, agg], axis=-1)
    g1 = jnp.maximum(nin @ nW1_ref[...] + nb1_ref[...], 0.0)
    g2 = _ln64(g1 @ nW2_ref[...] + nb2_ref[...], ng_ref[...], nbeta_ref[...])
    out = jnp.maximum(g2, 0.0) @ nW3_ref[...] + nb3_ref[...]
    out_ref[...] = out.reshape(_BT, _N, _D_OUT)


@functools.partial(jax.jit, static_argnames=("interpret",))
def _run(states, eW1, eb1, eW2, eb2, eg, ebeta, eW3, eb3,
         nW1, nb1, nW2, nb2, ng, nbeta, nW3, nb3, interpret=False):
    f32 = jnp.float32
    row = lambda v: v.reshape(1, -1)
    eW1a, eW1b = eW1[:_D_IN], eW1[_D_IN:]
    eW1a2 = jnp.concatenate([eW1a, eW1a], axis=1)
    z = jnp.zeros((_D_H, _D_H), f32)
    W2d = jnp.block([[eW2, z], [z, eW2]])
    b2d = row(jnp.concatenate([eb2, eb2]))
    ones = jnp.ones((_D_H, _D_H), f32) / _D_H
    A = jnp.block([[ones, z], [z, ones]])
    gd = jnp.concatenate([eg, eg])
    Wc = (W2d - W2d @ A) * gd[None, :]
    bc = (b2d - b2d @ A) * gd[None, :]
    Ag = A / (gd * gd)[:, None]
    betad = row(jnp.concatenate([ebeta, ebeta]))
    W3v = jnp.concatenate([eW3, eW3], axis=0)
    weights = (eW1a2, eW1a, eW1b, row(eb1), Wc, bc, Ag,
               betad, W3v, row(eb3), eW2, row(eb2), row(eg), row(ebeta),
               eW3, nW1, row(nb1), nW2, row(nb2), row(ng), row(nbeta),
               nW3, row(nb3))
    full = lambda w: pl.BlockSpec(w.shape, lambda b: (0,) * w.ndim)
    out = pl.pallas_call(
        _fused_kernel,
        grid=(_B // _BT,),
        in_specs=[pl.BlockSpec((_BT, _N, _D_IN), lambda b: (b, 0, 0))]
                 + [full(w) for w in weights],
        out_specs=pl.BlockSpec((_BT, _N, _D_OUT), lambda b: (b, 0, 0)),
        out_shape=jax.ShapeDtypeStruct((_B, _N, _D_OUT), f32),
        compiler_params=pltpu.CompilerParams(
            dimension_semantics=("parallel",)),
        interpret=interpret,
    )(states, *weights)
    return out


def kernel(states, action, viz, eW1, eb1, eW2, eb2, eg, ebeta, eW3, eb3,
           nW1, nb1, nW2, nb2, ng, nbeta, nW3, nb3):
    out = _run(states, eW1, eb1, eW2, eb2, eg, ebeta, eW3, eb3,
               nW1, nb1, nW2, nb2, ng, nbeta, nW3, nb3)
    return (out, action, viz)
